# initial kernel scaffold (unmeasured)
import functools

import jax
import jax.numpy as jnp
from jax import lax
from jax.experimental import pallas as pl
from jax.experimental.pallas import tpu as pltpu

N_EXPERTS = 8
E_LOCAL = 4
C = 640


def _moe_body(x_local_ref, x_send_ref, w1_ref, w2_ref,
              out_local_ref, out_recv_ref,
              recv_x, y_send,
              d_send_sems, d_recv_sems, c_send_sems, c_recv_sems):
    j = pl.program_id(0)
    my_x = lax.axis_index("x")
    my_y = lax.axis_index("y")
    my_z = lax.axis_index("z")
    peer = (1 - my_x, my_y, my_z)

    def dispatch_rdma(idx):
        return pltpu.make_async_remote_copy(
            src_ref=x_send_ref.at[idx],
            dst_ref=recv_x.at[idx],
            send_sem=d_send_sems.at[idx],
            recv_sem=d_recv_sems.at[idx],
            device_id=peer,
            device_id_type=pl.DeviceIdType.MESH,
        )

    def combine_rdma(idx):
        return pltpu.make_async_remote_copy(
            src_ref=y_send.at[idx],
            dst_ref=out_recv_ref.at[idx],
            send_sem=c_send_sems.at[idx],
            recv_sem=c_recv_sems.at[idx],
            device_id=peer,
            device_id_type=pl.DeviceIdType.MESH,
        )

    @pl.when(j == 0)
    def _():
        barrier_sem = pltpu.get_barrier_semaphore()
        pl.semaphore_signal(
            barrier_sem, inc=1, device_id=peer,
            device_id_type=pl.DeviceIdType.MESH,
        )
        pl.semaphore_wait(barrier_sem, 1)
        for jj in range(E_LOCAL):
            dispatch_rdma(jj).start()

    dispatch_rdma(j).wait_recv()

    w1 = w1_ref[0]
    w2 = w2_ref[0]

    xl = x_local_ref[0]
    h = jnp.maximum(jnp.dot(xl, w1, preferred_element_type=jnp.float32), 0.0)
    yl = jnp.dot(h.astype(jnp.bfloat16), w2,
                 preferred_element_type=jnp.float32)
    out_local_ref[0] = yl.astype(jnp.bfloat16)

    xr = recv_x[pl.ds(j, 1)].reshape(C, -1)
    hr = jnp.maximum(jnp.dot(xr, w1, preferred_element_type=jnp.float32), 0.0)
    yr = jnp.dot(hr.astype(jnp.bfloat16), w2,
                 preferred_element_type=jnp.float32)
    y_send[pl.ds(j, 1)] = yr.astype(jnp.bfloat16)[None]
    combine_rdma(j).start()

    @pl.when(j == E_LOCAL - 1)
    def _():
        for jj in range(E_LOCAL):
            dispatch_rdma(jj).wait_send()
            combine_rdma(jj).wait_send()
            combine_rdma(jj).wait_recv()


def kernel(x, assign, W1, W2):
    T, d = x.shape
    f = W1.shape[2]
    my_x = lax.axis_index("x")

    sort_idx = jnp.argsort(assign)
    a_sorted = assign[sort_idx]
    counts = jnp.bincount(assign, length=N_EXPERTS)
    starts = jnp.cumsum(counts) - counts
    rank = jnp.arange(T, dtype=jnp.int32) - starts[a_sorted].astype(jnp.int32)
    slot = a_sorted * C + rank
    slot = jnp.where(rank < C, slot, N_EXPERTS * C)
    x_sorted = x[sort_idx].astype(jnp.bfloat16)
    xp = jnp.zeros((N_EXPERTS * C, d), jnp.bfloat16)
    xp = xp.at[slot].set(x_sorted, mode="drop").reshape(N_EXPERTS, C, d)

    x_local = lax.dynamic_slice(xp, (E_LOCAL * my_x, 0, 0), (E_LOCAL, C, d))
    x_send = lax.dynamic_slice(xp, (E_LOCAL * (1 - my_x), 0, 0),
                               (E_LOCAL, C, d))

    w1 = W1.astype(jnp.bfloat16)
    w2 = W2.astype(jnp.bfloat16)

    out_local, out_recv = pl.pallas_call(
        _moe_body,
        grid=(E_LOCAL,),
        in_specs=[
            pl.BlockSpec((1, C, d), lambda j: (j, 0, 0)),
            pl.BlockSpec(memory_space=pltpu.ANY),
            pl.BlockSpec((1, d, f), lambda j: (j, 0, 0)),
            pl.BlockSpec((1, f, d), lambda j: (j, 0, 0)),
        ],
        out_specs=[
            pl.BlockSpec((1, C, d), lambda j: (j, 0, 0)),
            pl.BlockSpec(memory_space=pltpu.ANY),
        ],
        out_shape=[
            jax.ShapeDtypeStruct((E_LOCAL, C, d), jnp.bfloat16),
            jax.ShapeDtypeStruct((E_LOCAL, C, d), jnp.bfloat16),
        ],
        scratch_shapes=[
            pltpu.VMEM((E_LOCAL, C, d), jnp.bfloat16),
            pltpu.VMEM((E_LOCAL, C, d), jnp.bfloat16),
            pltpu.SemaphoreType.DMA((E_LOCAL,)),
            pltpu.SemaphoreType.DMA((E_LOCAL,)),
            pltpu.SemaphoreType.DMA((E_LOCAL,)),
            pltpu.SemaphoreType.DMA((E_LOCAL,)),
        ],
        compiler_params=pltpu.CompilerParams(collective_id=0),
    )(x_local, x_send, w1, w2)

    y8 = jnp.zeros((N_EXPERTS, C, d), jnp.bfloat16)
    y8 = lax.dynamic_update_slice(y8, out_local, (E_LOCAL * my_x, 0, 0))
    y8 = lax.dynamic_update_slice(y8, out_recv, (E_LOCAL * (1 - my_x), 0, 0))
    y_flat = y8.reshape(N_EXPERTS * C, d)
    out_sorted = y_flat[jnp.clip(slot, 0, N_EXPERTS * C - 1)]
    out = jnp.zeros((T, d), jnp.bfloat16).at[sort_idx].set(out_sorted)
    return out.astype(jnp.float32)


# baseline (device time: 892102 ns/iter reference)
import jax
import jax.numpy as jnp
from jax import lax
from jax.experimental import pallas as pl
from jax.experimental.pallas import tpu as pltpu

N_EXPERTS = 8
E_LOCAL = 4
C = 576
F_TILES = 4


def _moe_body(x_local_ref, x_send_ref, w1_ref, w2_ref,
              out_local_ref, out_recv_ref,
              recv_x, y_send,
              acc_l, acc_r,
              d_send_sem, d_recv_sem, c_send_sem, c_recv_sems):
    j = pl.program_id(0)
    k = pl.program_id(1)
    my_x = lax.axis_index("x")
    my_y = lax.axis_index("y")
    my_z = lax.axis_index("z")
    peer = (1 - my_x, my_y, my_z)

    def dispatch_rdma():
        return pltpu.make_async_remote_copy(
            src_ref=x_send_ref,
            dst_ref=recv_x,
            send_sem=d_send_sem,
            recv_sem=d_recv_sem,
            device_id=peer,
            device_id_type=pl.DeviceIdType.MESH,
        )

    def combine_rdma(idx):
        return pltpu.make_async_remote_copy(
            src_ref=y_send,
            dst_ref=out_recv_ref.at[idx],
            send_sem=c_send_sem,
            recv_sem=c_recv_sems.at[idx],
            device_id=peer,
            device_id_type=pl.DeviceIdType.MESH,
        )

    @pl.when(jnp.logical_and(j == 0, k == 0))
    def _():
        barrier_sem = pltpu.get_barrier_semaphore()
        pl.semaphore_signal(
            barrier_sem, inc=1, device_id=peer,
            device_id_type=pl.DeviceIdType.MESH,
        )
        pl.semaphore_wait(barrier_sem, 1)
        rdma = dispatch_rdma()
        rdma.start()
        rdma.wait()

    w1t = w1_ref[0]
    w2t = w2_ref[0]
    xl = x_local_ref[0]
    xr = recv_x[pl.ds(j, 1)].reshape(C, -1)

    hl = jnp.maximum(jnp.dot(xl, w1t, preferred_element_type=jnp.float32), 0.0)
    pl_part = jnp.dot(hl.astype(jnp.bfloat16), w2t,
                      preferred_element_type=jnp.float32)
    hr = jnp.maximum(jnp.dot(xr, w1t, preferred_element_type=jnp.float32), 0.0)
    pr_part = jnp.dot(hr.astype(jnp.bfloat16), w2t,
                      preferred_element_type=jnp.float32)

    @pl.when(k == 0)
    def _():
        acc_l[...] = pl_part
        acc_r[...] = pr_part

    @pl.when(k > 0)
    def _():
        acc_l[...] += pl_part
        acc_r[...] += pr_part

    @pl.when(k == F_TILES - 1)
    def _():
        out_local_ref[0] = acc_l[...].astype(jnp.bfloat16)
        y_send[...] = acc_r[...].astype(jnp.bfloat16)
        for jj in range(E_LOCAL):
            @pl.when(j == jj)
            def _(jj=jj):
                rdma = combine_rdma(jj)
                rdma.start()
                rdma.wait_send()

    @pl.when(jnp.logical_and(j == E_LOCAL - 1, k == F_TILES - 1))
    def _():
        for jj in range(E_LOCAL):
            combine_rdma(jj).wait_recv()


def kernel(x, assign, W1, W2):
    T, d = x.shape
    f = W1.shape[2]
    ft = f // F_TILES
    my_x = lax.axis_index("x")

    sort_idx = jnp.argsort(assign)
    a_sorted = assign[sort_idx]
    counts = jnp.bincount(assign, length=N_EXPERTS)
    starts = jnp.cumsum(counts) - counts
    rank = jnp.arange(T, dtype=jnp.int32) - starts[a_sorted].astype(jnp.int32)
    slot = a_sorted * C + rank
    slot = jnp.where(rank < C, slot, N_EXPERTS * C)
    x_sorted = x[sort_idx].astype(jnp.bfloat16)
    xp = jnp.zeros((N_EXPERTS * C, d), jnp.bfloat16)
    xp = xp.at[slot].set(x_sorted, mode="drop").reshape(N_EXPERTS, C, d)

    x_local = lax.dynamic_slice(xp, (E_LOCAL * my_x, 0, 0), (E_LOCAL, C, d))
    x_send = lax.dynamic_slice(xp, (E_LOCAL * (1 - my_x), 0, 0),
                               (E_LOCAL, C, d))

    w1 = W1.astype(jnp.bfloat16)
    w2 = W2.astype(jnp.bfloat16)

    out_local, out_recv = pl.pallas_call(
        _moe_body,
        grid=(E_LOCAL, F_TILES),
        in_specs=[
            pl.BlockSpec((1, C, d), lambda j, k: (j, 0, 0)),
            pl.BlockSpec(memory_space=pl.ANY),
            pl.BlockSpec((1, d, ft), lambda j, k: (j, 0, k)),
            pl.BlockSpec((1, ft, d), lambda j, k: (j, k, 0)),
        ],
        out_specs=[
            pl.BlockSpec((1, C, d), lambda j, k: (j, 0, 0)),
            pl.BlockSpec(memory_space=pl.ANY),
        ],
        out_shape=[
            jax.ShapeDtypeStruct((E_LOCAL, C, d), jnp.bfloat16),
            jax.ShapeDtypeStruct((E_LOCAL, C, d), jnp.bfloat16),
        ],
        scratch_shapes=[
            pltpu.VMEM((E_LOCAL, C, d), jnp.bfloat16),
            pltpu.VMEM((C, d), jnp.bfloat16),
            pltpu.VMEM((C, d), jnp.float32),
            pltpu.VMEM((C, d), jnp.float32),
            pltpu.SemaphoreType.DMA,
            pltpu.SemaphoreType.DMA,
            pltpu.SemaphoreType.DMA,
            pltpu.SemaphoreType.DMA((E_LOCAL,)),
        ],
        compiler_params=pltpu.CompilerParams(
            collective_id=0,
            vmem_limit_bytes=63 * 1024 * 1024,
        ),
    )(x_local, x_send, w1, w2)

    y8 = jnp.zeros((N_EXPERTS, C, d), jnp.bfloat16)
    y8 = lax.dynamic_update_slice(y8, out_local, (E_LOCAL * my_x, 0, 0))
    y8 = lax.dynamic_update_slice(y8, out_recv, (E_LOCAL * (1 - my_x), 0, 0))
    y_flat = y8.reshape(N_EXPERTS * C, d)
    out_sorted = y_flat[jnp.clip(slot, 0, N_EXPERTS * C - 1)]
    out = jnp.zeros((T, d), jnp.bfloat16).at[sort_idx].set(out_sorted)
    return out.astype(jnp.float32)


# device time: 451846 ns/iter; 1.9743x vs baseline; 1.9743x over previous
import jax
import jax.numpy as jnp
from jax import lax
from jax.experimental import pallas as pl
from jax.experimental.pallas import tpu as pltpu

N_EXPERTS = 8
E_LOCAL = 4
C = 576
F_TILES = 8


def _moe_body(xp_ref, w1_ref, w2_ref, out_ref,
              recv_x, xl, y_stage, y_send, acc_l, acc_r,
              d_send_sems, d_recv_sems, c_send_sems, c_recv_sems,
              lx_sem, lo_sem):
    j = pl.program_id(0)
    k = pl.program_id(1)
    my_x = lax.axis_index("x")
    my_y = lax.axis_index("y")
    my_z = lax.axis_index("z")
    peer = (1 - my_x, my_y, my_z)

    def dispatch_rdma(jj, sx):
        return pltpu.make_async_remote_copy(
            src_ref=xp_ref.at[E_LOCAL * (1 - sx) + jj],
            dst_ref=recv_x.at[jj],
            send_sem=d_send_sems.at[jj],
            recv_sem=d_recv_sems.at[jj],
            device_id=peer,
            device_id_type=pl.DeviceIdType.MESH,
        )

    def combine_rdma(jj, sx):
        return pltpu.make_async_remote_copy(
            src_ref=y_send.at[jj % 2],
            dst_ref=out_ref.at[E_LOCAL * sx + jj],
            send_sem=c_send_sems.at[jj],
            recv_sem=c_recv_sems.at[jj],
            device_id=peer,
            device_id_type=pl.DeviceIdType.MESH,
        )

    def for_my_x(fn):
        for sx in (0, 1):
            @pl.when(my_x == sx)
            def _(sx=sx):
                fn(sx)

    @pl.when(jnp.logical_and(j == 0, k == 0))
    def _():
        barrier_sem = pltpu.get_barrier_semaphore()
        pl.semaphore_signal(
            barrier_sem, inc=1, device_id=peer,
            device_id_type=pl.DeviceIdType.MESH,
        )
        pl.semaphore_wait(barrier_sem, 1)

        def start_dispatch(sx):
            for jj in range(E_LOCAL):
                dispatch_rdma(jj, sx).start()
        for_my_x(start_dispatch)

    @pl.when(k == 0)
    def _():
        for jj in range(E_LOCAL):
            @pl.when(j == jj)
            def _(jj=jj):
                def load_local(sx):
                    cp = pltpu.make_async_copy(
                        xp_ref.at[E_LOCAL * sx + jj], xl, lx_sem)
                    cp.start()
                    cp.wait()
                for_my_x(load_local)
                dispatch_rdma(jj, 0).wait_recv()

    w1t = w1_ref[0].astype(jnp.bfloat16)
    w2t = w2_ref[0].astype(jnp.bfloat16)
    xlv = xl[...]
    xr = recv_x[pl.ds(j, 1)].reshape(xlv.shape)

    hl = jnp.maximum(jnp.dot(xlv, w1t, preferred_element_type=jnp.float32), 0.0)
    pl_part = jnp.dot(hl.astype(jnp.bfloat16), w2t,
                      preferred_element_type=jnp.float32)
    hr = jnp.maximum(jnp.dot(xr, w1t, preferred_element_type=jnp.float32), 0.0)
    pr_part = jnp.dot(hr.astype(jnp.bfloat16), w2t,
                      preferred_element_type=jnp.float32)

    @pl.when(k == 0)
    def _():
        acc_l[...] = pl_part
        acc_r[...] = pr_part

    @pl.when(k > 0)
    def _():
        acc_l[...] += pl_part
        acc_r[...] += pr_part

    @pl.when(k == F_TILES - 1)
    def _():
        for jj in range(E_LOCAL):
            @pl.when(j == jj)
            def _(jj=jj):
                y_stage[...] = acc_l[...].astype(jnp.bfloat16)

                def store_local(sx):
                    cp = pltpu.make_async_copy(
                        y_stage, out_ref.at[E_LOCAL * sx + jj], lo_sem)
                    cp.start()
                    cp.wait()
                for_my_x(store_local)

                if jj >= 2:
                    for_my_x(lambda sx, jj=jj: combine_rdma(jj - 2, sx).wait_send())
                y_send[jj % 2] = acc_r[...].astype(jnp.bfloat16)
                for_my_x(lambda sx, jj=jj: combine_rdma(jj, sx).start())

    @pl.when(jnp.logical_and(j == E_LOCAL - 1, k == F_TILES - 1))
    def _():
        def drain(sx):
            for jj in range(E_LOCAL):
                dispatch_rdma(jj, sx).wait_send()
            for jj in range(E_LOCAL - 2, E_LOCAL):
                combine_rdma(jj, sx).wait_send()
            for jj in range(E_LOCAL):
                pltpu.make_async_remote_copy(
                    src_ref=y_send.at[jj % 2],
                    dst_ref=out_ref.at[E_LOCAL * (1 - sx) + jj],
                    send_sem=c_send_sems.at[jj],
                    recv_sem=c_recv_sems.at[jj],
                    device_id=peer,
                    device_id_type=pl.DeviceIdType.MESH,
                ).wait_recv()
        for_my_x(drain)


def kernel(x, assign, W1, W2):
    T, d = x.shape
    f = W1.shape[2]
    ft = f // F_TILES
    x16 = x.astype(jnp.bfloat16)

    sort_idx = jnp.argsort(assign)
    a_sorted = assign[sort_idx]
    counts = jnp.bincount(assign, length=N_EXPERTS)
    starts = jnp.cumsum(counts) - counts
    pos = starts[:, None] + jnp.arange(C, dtype=jnp.int32)[None, :]
    tok = sort_idx[jnp.clip(pos, 0, T - 1)]
    xp = x16[tok]

    out8 = pl.pallas_call(
        _moe_body,
        grid=(E_LOCAL, F_TILES),
        in_specs=[
            pl.BlockSpec(memory_space=pl.ANY),
            pl.BlockSpec((1, d, ft), lambda j, k: (j, 0, k)),
            pl.BlockSpec((1, ft, d), lambda j, k: (j, k, 0)),
        ],
        out_specs=pl.BlockSpec(memory_space=pl.ANY),
        out_shape=jax.ShapeDtypeStruct((N_EXPERTS, C, d), jnp.bfloat16),
        scratch_shapes=[
            pltpu.VMEM((E_LOCAL, C, d), jnp.bfloat16),
            pltpu.VMEM((C, d), jnp.bfloat16),
            pltpu.VMEM((C, d), jnp.bfloat16),
            pltpu.VMEM((2, C, d), jnp.bfloat16),
            pltpu.VMEM((C, d), jnp.float32),
            pltpu.VMEM((C, d), jnp.float32),
            pltpu.SemaphoreType.DMA((E_LOCAL,)),
            pltpu.SemaphoreType.DMA((E_LOCAL,)),
            pltpu.SemaphoreType.DMA((E_LOCAL,)),
            pltpu.SemaphoreType.DMA((E_LOCAL,)),
            pltpu.SemaphoreType.DMA,
            pltpu.SemaphoreType.DMA,
        ],
        compiler_params=pltpu.CompilerParams(
            collective_id=0,
            vmem_limit_bytes=63 * 1024 * 1024,
        ),
    )(xp, W1, W2)

    rank = jnp.arange(T, dtype=jnp.int32) - starts[a_sorted].astype(jnp.int32)
    slot_sorted = a_sorted * C + jnp.minimum(rank, C - 1)
    slot_by_token = jnp.zeros((T,), jnp.int32).at[sort_idx].set(slot_sorted)
    out = out8.reshape(N_EXPERTS * C, d)[slot_by_token]
    return out.astype(jnp.float32)
